# Initial kernel scaffold; baseline (speedup 1.0000x reference)
import jax
import jax.numpy as jnp
from jax.experimental import pallas as pl


def kernel(x, mag_table, ori_w, cell_w, am_w):
    raise NotImplementedError("write your pallas kernel here")


# speedup vs baseline: 9.8537x; 9.8537x over previous
import jax
import jax.numpy as jnp
from jax import lax
from jax.experimental import pallas as pl

DIM = 8192
CELLS = 576
LEVELS = 256
BINS = 9
PAIRS = CELLS * BINS
PAIRS_PAD = 5248
CHUNK = 512
D_STEPS = DIM // CHUNK


def _thresh_body(mag_ref, xr_ref, c_ref):
    rowsum = jnp.sum(mag_ref[...], axis=1)
    counts = (rowsum + float(DIM)) * 0.5
    xr = xr_ref[...]
    idx = jnp.clip(jnp.round(xr * float(LEVELS - 1)), 0.0, float(LEVELS - 1))
    idx = idx.astype(jnp.int32)
    lvl = lax.broadcasted_iota(jnp.int32, (1, LEVELS), 1)
    onehot = idx == lvl
    c = jnp.sum(jnp.where(onehot, counts[None, :], 0.0), axis=1, keepdims=True)
    c_ref[...] = c


def _main_body(cth_ref, ori_ref, cw_ref, am_ref, out_ref):
    j = pl.program_id(0)
    dvec = (lax.broadcasted_iota(jnp.int32, (1, CHUNK), 1)
            + j * CHUNK).astype(jnp.float32)
    cw = cw_ref[...]
    ori = ori_ref[...]
    cth = cth_ref[...]
    acc = jnp.zeros((1, CHUNK), jnp.float32)
    for b in range(BINS):
        mask = dvec < cth[:, b:b + 1]
        a_b = jnp.sum(jnp.where(mask, cw, 0.0), axis=0, keepdims=True)
        acc = acc + ori[b:b + 1, :] * a_b
    s_col = jnp.sum(ori, axis=0, keepdims=True)
    c_col = jnp.sum(cw, axis=0, keepdims=True)
    mat = 2.0 * acc - s_col * c_col
    partial = jnp.sum(am_ref[...] * mat, axis=1, keepdims=True)

    @pl.when(j == 0)
    def _():
        out_ref[...] = partial

    @pl.when(j > 0)
    def _():
        out_ref[...] = out_ref[...] + partial


def kernel(x, mag_table, ori_w, cell_w, am_w):
    num_classes = am_w.shape[0]
    xf = jnp.reshape(x, (PAIRS, 1))
    xf = jnp.pad(xf, ((0, PAIRS_PAD - PAIRS), (0, 0)))
    c = pl.pallas_call(
        _thresh_body,
        out_shape=jax.ShapeDtypeStruct((PAIRS_PAD, 1), jnp.float32),
    )(mag_table, xf)
    cth = jnp.reshape(c[:PAIRS, 0], (CELLS, BINS))
    scores = pl.pallas_call(
        _main_body,
        grid=(D_STEPS,),
        in_specs=[
            pl.BlockSpec((CELLS, BINS), lambda j: (0, 0)),
            pl.BlockSpec((BINS, CHUNK), lambda j: (0, j)),
            pl.BlockSpec((CELLS, CHUNK), lambda j: (0, j)),
            pl.BlockSpec((num_classes, CHUNK), lambda j: (0, j)),
        ],
        out_specs=pl.BlockSpec((num_classes, 1), lambda j: (0, 0)),
        out_shape=jax.ShapeDtypeStruct((num_classes, 1), jnp.float32),
    )(cth, ori_w, cell_w, am_w)
    return jnp.reshape(scores, (num_classes,))
